# Initial kernel scaffold; baseline (speedup 1.0000x reference)
import jax
import jax.numpy as jnp
from jax.experimental import pallas as pl


def kernel(node_features, edge_index, edge_features, W1, b1, W2, b2, U1, ub1, U2, ub2, gamma, beta):
    raise NotImplementedError("write your pallas kernel here")


# speedup vs baseline: 3.3521x; 3.3521x over previous
import functools

import jax
import jax.numpy as jnp
from jax import lax
from jax.experimental import pallas as pl
from jax.experimental.pallas import tpu as pltpu
from jax.experimental.pallas import tpu_sc as plsc

N = 10000
E = 320000
ND = 128
ED = 16
H = 128

NC = 2
NS = 16
NW = NC * NS
EPW = E // NW
BLK = 80
NBLK = EPW // BLK
NPT = 624
NTAIL = N - NPT * NS

_SQRT_HALF = 0.7071067811865476


def _gelu(x):
    return 0.5 * x * (1.0 + lax.erf(x * _SQRT_HALF))


def _pre_body(x_ref, w1a_ref, w1b_ref, a_ref, b_ref):
    x = x_ref[...]
    a_ref[...] = jnp.dot(x, w1a_ref[...], preferred_element_type=jnp.float32)
    b_ref[...] = jnp.dot(x, w1b_ref[...], preferred_element_type=jnp.float32)


def _tc_pre(x, w1a, w1b):
    return pl.pallas_call(
        _pre_body,
        out_shape=(
            jax.ShapeDtypeStruct((N, H), jnp.float32),
            jax.ShapeDtypeStruct((N, H), jnp.float32),
        ),
    )(x, w1a, w1b)


def _sc_gather_body(a_hbm, b_hbm, src_hbm, dst_hbm, out_hbm,
                    sidx, didx, arows, brows, sema, semb):
    wid = lax.axis_index("s") * NC + lax.axis_index("c")
    base = wid * EPW

    def block(i, carry):
        off = base + i * BLK
        pltpu.sync_copy(src_hbm.at[pl.ds(off, BLK)], sidx)
        pltpu.sync_copy(dst_hbm.at[pl.ds(off, BLK)], didx)
        ca = pltpu.async_copy(a_hbm.at[sidx], arows, sema)
        cb = pltpu.async_copy(b_hbm.at[didx], brows, semb)
        ca.wait()
        cb.wait()

        def row(e, c2):
            for j in range(H // 16):
                sl = pl.ds(j * 16, 16)
                arows[e, sl] = arows[e, sl] + brows[e, sl]
            return c2

        lax.fori_loop(0, BLK, row, 0)
        pltpu.sync_copy(arows, out_hbm.at[pl.ds(off, BLK)])
        return carry

    lax.fori_loop(0, NBLK, block, 0)


_sc_gather = functools.partial(
    pl.kernel,
    out_type=jax.ShapeDtypeStruct((E, H), jnp.float32),
    mesh=plsc.VectorSubcoreMesh(core_axis_name="c", subcore_axis_name="s"),
    scratch_types=[
        pltpu.VMEM((BLK,), jnp.int32),
        pltpu.VMEM((BLK,), jnp.int32),
        pltpu.VMEM((BLK, H), jnp.float32),
        pltpu.VMEM((BLK, H), jnp.float32),
        pltpu.SemaphoreType.DMA,
        pltpu.SemaphoreType.DMA,
    ],
)(_sc_gather_body)


TB = 2560


def _msg_body(g_ref, ef_ref, w1c_ref, b1_ref, w2_ref, b2_ref, m_ref):
    h = (g_ref[...]
         + jnp.dot(ef_ref[...], w1c_ref[...], preferred_element_type=jnp.float32)
         + b1_ref[...])
    h = _gelu(h)
    m = jnp.dot(h, w2_ref[...], preferred_element_type=jnp.float32) + b2_ref[...]
    m_ref[...] = _gelu(m)


def _tc_msg(g, ef, w1c, b1, w2, b2):
    return pl.pallas_call(
        _msg_body,
        grid=(E // TB,),
        in_specs=[
            pl.BlockSpec((TB, H), lambda i: (i, 0)),
            pl.BlockSpec((TB, ED), lambda i: (i, 0)),
            pl.BlockSpec((ED, H), lambda i: (0, 0)),
            pl.BlockSpec((1, H), lambda i: (0, 0)),
            pl.BlockSpec((H, H), lambda i: (0, 0)),
            pl.BlockSpec((1, H), lambda i: (0, 0)),
        ],
        out_specs=pl.BlockSpec((TB, H), lambda i: (i, 0)),
        out_shape=jax.ShapeDtypeStruct((E, H), jnp.float32),
        compiler_params=pltpu.CompilerParams(
            dimension_semantics=("arbitrary",)),
    )(g, ef, w1c, b1.reshape(1, H), w2, b2.reshape(1, H))


def _sc_scatter_body(m_hbm, dst_hbm, zero_hbm, out_hbm,
                     didx, mrows, agg_sh):
    c = lax.axis_index("c")
    s = lax.axis_index("s")
    wid = s * NC + c
    base = wid * EPW

    pltpu.sync_copy(zero_hbm.at[pl.ds(s * NPT, NPT)],
                    agg_sh.at[pl.ds(s * NPT, NPT)])

    @pl.when(s == NS - 1)
    def _zero_tail():
        pltpu.sync_copy(zero_hbm.at[pl.ds(NPT * NS, NTAIL)],
                        agg_sh.at[pl.ds(NPT * NS, NTAIL)])

    plsc.subcore_barrier()

    def block(i, carry):
        off = base + i * BLK
        pltpu.sync_copy(dst_hbm.at[pl.ds(off, BLK)], didx)
        pltpu.sync_copy(m_hbm.at[pl.ds(off, BLK)], mrows)
        pltpu.sync_copy(mrows, agg_sh.at[didx], add=True)
        return carry

    lax.fori_loop(0, NBLK, block, 0)
    plsc.subcore_barrier()
    pltpu.sync_copy(agg_sh.at[pl.ds(s * NPT, NPT)],
                    out_hbm.at[c, pl.ds(s * NPT, NPT)])

    @pl.when(s == NS - 1)
    def _out_tail():
        pltpu.sync_copy(agg_sh.at[pl.ds(NPT * NS, NTAIL)],
                        out_hbm.at[c, pl.ds(NPT * NS, NTAIL)])


_sc_scatter = functools.partial(
    pl.kernel,
    out_type=jax.ShapeDtypeStruct((NC, N, H), jnp.float32),
    mesh=plsc.VectorSubcoreMesh(core_axis_name="c", subcore_axis_name="s"),
    scratch_types=[
        pltpu.VMEM((BLK,), jnp.int32),
        pltpu.VMEM((BLK, H), jnp.float32),
        pltpu.VMEM_SHARED((N, H), jnp.float32),
    ],
)(_sc_scatter_body)


def _upd_body(x_ref, p_ref, u1a_ref, u1b_ref, ub1_ref, u2_ref, ub2_ref,
              gamma_ref, beta_ref, o_ref):
    x = x_ref[...]
    agg = p_ref[0] + p_ref[1]
    u = (jnp.dot(x, u1a_ref[...], preferred_element_type=jnp.float32)
         + jnp.dot(agg, u1b_ref[...], preferred_element_type=jnp.float32)
         + ub1_ref[...])
    u = _gelu(u)
    y = x + jnp.dot(u, u2_ref[...], preferred_element_type=jnp.float32) + ub2_ref[...]
    mu = jnp.mean(y, axis=-1, keepdims=True)
    var = jnp.mean((y - mu) ** 2, axis=-1, keepdims=True)
    o_ref[...] = (y - mu) * lax.rsqrt(var + 1e-5) * gamma_ref[...] + beta_ref[...]


def _tc_upd(x, p, u1a, u1b, ub1, u2, ub2, gamma, beta):
    return pl.pallas_call(
        _upd_body,
        out_shape=jax.ShapeDtypeStruct((N, ND), jnp.float32),
    )(x, p, u1a, u1b, ub1.reshape(1, H), u2, ub2.reshape(1, ND),
      gamma.reshape(1, ND), beta.reshape(1, ND))


def kernel(node_features, edge_index, edge_features,
           W1, b1, W2, b2, U1, ub1, U2, ub2, gamma, beta):
    src = edge_index[0].astype(jnp.int32)
    dst = edge_index[1].astype(jnp.int32)
    w1a = W1[:ND]
    w1b = W1[ND:2 * ND]
    w1c = W1[2 * ND:]

    a, b = _tc_pre(node_features, w1a, w1b)
    g = _sc_gather(a, b, src, dst)
    m = _tc_msg(g, edge_features, w1c, b1, W2, b2)
    zeros = jnp.zeros((N, H), dtype=jnp.float32)
    partials = _sc_scatter(m, dst, zeros)
    return _tc_upd(node_features, partials, U1[:ND], U1[ND:], ub1,
                   U2, ub2, gamma, beta)
